# baseline (speedup 1.0000x reference)
import functools

import jax
import jax.numpy as jnp
from jax import lax
from jax.experimental import pallas as pl
from jax.experimental.pallas import tpu as pltpu
from jax.experimental.pallas import tpu_sc as plsc

N = 10000
NPAD = 10240
E = 320000
D_IN = 128
D_HID = 128
N_CLS = 64

NW = 32
CHUNK = 64
NCH = 160
NHALF = 2
HALF = NCH // NHALF
NBUF = 4
EPAD = NW * NCH * CHUNK
RPS = NPAD // 16

_mesh = plsc.VectorSubcoreMesh(core_axis_name="c", subcore_axis_name="s")
_sc_params = pltpu.CompilerParams(use_tc_tiling_on_sc=False)


def _make_agg(D):
  @functools.partial(
      pl.kernel,
      out_type=jax.ShapeDtypeStruct((2, NPAD, D), jnp.float32),
      mesh=_mesh,
      scratch_types=[
          pltpu.VMEM((HALF, CHUNK), jnp.int32),
          pltpu.VMEM((HALF, CHUNK), jnp.int32),
          [pltpu.VMEM((CHUNK, D), jnp.float32) for _ in range(NBUF)],
          pltpu.VMEM_SHARED((NPAD, D), jnp.float32),
          [pltpu.SemaphoreType.DMA for _ in range(NBUF)],
      ],
      compiler_params=_sc_params,
  )
  def agg(g_hbm, src_hbm, dst_hbm, zeros_hbm, out_hbm,
          src_v, dst_v, bufs, acc_sh, sems):
    c = lax.axis_index("c")
    s = lax.axis_index("s")
    wid = s * 2 + c
    pltpu.sync_copy(zeros_hbm, acc_sh.at[pl.ds(s * RPS, RPS)])
    plsc.subcore_barrier()

    def body(st, carry):
      for b in range(NBUF):
        ch = st * NBUF + b
        pltpu.make_async_copy(g_hbm.at[src_v.at[ch]], bufs[b], sems[b]).wait()
        pltpu.sync_copy(bufs[b], acc_sh.at[dst_v.at[ch]], add=True)
        nxt = ch + NBUF

        @pl.when(nxt < HALF)
        def _():
          pltpu.async_copy(g_hbm.at[src_v.at[nxt]], bufs[b], sems[b])

      return carry

    for h in range(NHALF):
      pltpu.sync_copy(src_hbm.at[wid, h], src_v)
      pltpu.sync_copy(dst_hbm.at[wid, h], dst_v)
      for b in range(NBUF):
        pltpu.async_copy(g_hbm.at[src_v.at[b]], bufs[b], sems[b])
      lax.fori_loop(0, HALF // NBUF, body, 0)
    plsc.subcore_barrier()
    pltpu.sync_copy(acc_sh.at[pl.ds(s * RPS, RPS)],
                    out_hbm.at[c, pl.ds(s * RPS, RPS)])

  return agg


_agg128 = _make_agg(D_HID)
_agg64 = _make_agg(N_CLS)


@functools.partial(
    pl.kernel,
    out_type=jax.ShapeDtypeStruct((2, NPAD, 16), jnp.float32),
    mesh=_mesh,
    scratch_types=[
        pltpu.VMEM((HALF, CHUNK), jnp.int32),
        pltpu.VMEM((CHUNK, 16), jnp.float32),
        pltpu.VMEM_SHARED((NPAD, 16), jnp.float32),
    ],
    compiler_params=_sc_params,
)
def _deg_kernel(dst_hbm, onehot_hbm, zeros_hbm, out_hbm, dst_v, ones_v, acc_sh):
  c = lax.axis_index("c")
  s = lax.axis_index("s")
  wid = s * 2 + c
  pltpu.sync_copy(zeros_hbm, acc_sh.at[pl.ds(s * RPS, RPS)])
  pltpu.sync_copy(onehot_hbm, ones_v)
  plsc.subcore_barrier()

  def body(ch, carry):
    pltpu.sync_copy(ones_v, acc_sh.at[dst_v.at[ch]], add=True)
    return carry

  for h in range(NHALF):
    pltpu.sync_copy(dst_hbm.at[wid, h], dst_v)
    lax.fori_loop(0, HALF, body, 0)
  plsc.subcore_barrier()
  pltpu.sync_copy(acc_sh.at[pl.ds(s * RPS, RPS)],
                  out_hbm.at[c, pl.ds(s * RPS, RPS)])


def _isd_of(da_ref, db_ref):
  deg = da_ref[:, 0:1] + db_ref[:, 0:1]
  return lax.rsqrt(jnp.maximum(deg, 1.0))


def _mm1_body(x_ref, w_ref, da_ref, db_ref, o_ref):
  isd = _isd_of(da_ref, db_ref)
  o_ref[...] = jnp.dot(x_ref[...], w_ref[...],
                       preferred_element_type=jnp.float32) * isd


def _mm2_body(aa_ref, ab_ref, da_ref, db_ref, b1_ref, w2_ref, o_ref):
  isd = _isd_of(da_ref, db_ref)
  h = jnp.maximum(isd * (aa_ref[...] + ab_ref[...]) + b1_ref[...], 0.0)
  o_ref[...] = jnp.dot(h, w2_ref[...],
                       preferred_element_type=jnp.float32) * isd


def _fin_body(aa_ref, ab_ref, da_ref, db_ref, b2_ref, o_ref):
  isd = _isd_of(da_ref, db_ref)
  o_ref[...] = isd * (aa_ref[...] + ab_ref[...]) + b2_ref[...]


_BLK = 512
_GRID = NPAD // _BLK


def _row_spec(d):
  return pl.BlockSpec((_BLK, d), lambda i: (i, 0))


def _full_spec(r, c):
  return pl.BlockSpec((r, c), lambda i: (0, 0))


_mm1 = pl.pallas_call(
    _mm1_body,
    grid=(_GRID,),
    in_specs=[_row_spec(D_IN), _full_spec(D_IN, D_HID),
              _row_spec(16), _row_spec(16)],
    out_specs=_row_spec(D_HID),
    out_shape=jax.ShapeDtypeStruct((NPAD, D_HID), jnp.float32),
)

_mm2 = pl.pallas_call(
    _mm2_body,
    grid=(_GRID,),
    in_specs=[_row_spec(D_HID), _row_spec(D_HID), _row_spec(16), _row_spec(16),
              _full_spec(1, D_HID), _full_spec(D_HID, N_CLS)],
    out_specs=_row_spec(N_CLS),
    out_shape=jax.ShapeDtypeStruct((NPAD, N_CLS), jnp.float32),
)

_fin = pl.pallas_call(
    _fin_body,
    grid=(_GRID,),
    in_specs=[_row_spec(N_CLS), _row_spec(N_CLS), _row_spec(16), _row_spec(16),
              _full_spec(1, N_CLS)],
    out_specs=_row_spec(N_CLS),
    out_shape=jax.ShapeDtypeStruct((NPAD, N_CLS), jnp.float32),
)


def kernel(x, edge_index, W1, b1, W2, b2):
  src = edge_index[0].astype(jnp.int32)
  dst = edge_index[1].astype(jnp.int32)
  pad = jnp.full((EPAD - E,), N, jnp.int32)
  src3 = jnp.concatenate([src, pad]).reshape(NW, NHALF, HALF, CHUNK)
  dst3 = jnp.concatenate([dst, pad]).reshape(NW, NHALF, HALF, CHUNK)
  xp = jnp.zeros((NPAD, D_IN), jnp.float32).at[:N].set(x)

  onehot = jnp.zeros((CHUNK, 16), jnp.float32).at[:, 0].set(1.0)
  z16 = jnp.zeros((RPS, 16), jnp.float32)
  z128 = jnp.zeros((RPS, D_HID), jnp.float32)
  z64 = jnp.zeros((RPS, N_CLS), jnp.float32)

  deg = _deg_kernel(dst3, onehot, z16)
  g1 = _mm1(xp, W1, deg[0], deg[1])
  agg1 = _agg128(g1, src3, dst3, z128)
  g2 = _mm2(agg1[0], agg1[1], deg[0], deg[1], b1.reshape(1, D_HID), W2)
  agg2 = _agg64(g2, src3, dst3, z64)
  out = _fin(agg2[0], agg2[1], deg[0], deg[1], b2.reshape(1, N_CLS))
  return out[:N]


# speedup vs baseline: 2.0581x; 2.0581x over previous
import functools

import jax
import jax.numpy as jnp
from jax import lax
from jax.experimental import pallas as pl
from jax.experimental.pallas import tpu as pltpu
from jax.experimental.pallas import tpu_sc as plsc

N = 10000
NPAD = 10240
E = 320000
D_IN = 128
D_HID = 128
N_CLS = 64

NW = 32
CHUNK = 64
NCH = 160
NHALF = 2
HALF = NCH // NHALF
NBUF = 4
EPAD = NW * NCH * CHUNK
RPS = NPAD // 16

_mesh = plsc.VectorSubcoreMesh(core_axis_name="c", subcore_axis_name="s")
_sc_params = pltpu.CompilerParams(use_tc_tiling_on_sc=False)


def _make_agg(D):
  @functools.partial(
      pl.kernel,
      out_type=jax.ShapeDtypeStruct((2, NPAD, D), jnp.float32),
      mesh=_mesh,
      scratch_types=[
          pltpu.VMEM((HALF, CHUNK), jnp.int32),
          pltpu.VMEM((HALF, CHUNK), jnp.int32),
          [pltpu.VMEM((CHUNK, D), jnp.float32) for _ in range(NBUF)],
          pltpu.VMEM_SHARED((NPAD, D), jnp.float32),
          pltpu.VMEM_SHARED((NPAD, D), jnp.float32),
          [pltpu.SemaphoreType.DMA for _ in range(NBUF)],
      ],
      compiler_params=_sc_params,
  )
  def agg(g_hbm, src_hbm, dst_hbm, zeros_hbm, out_hbm,
          src_v, dst_v, bufs, op_sh, acc_sh, sems):
    c = lax.axis_index("c")
    s = lax.axis_index("s")
    wid = s * 2 + c
    pltpu.sync_copy(g_hbm.at[pl.ds(s * RPS, RPS)], op_sh.at[pl.ds(s * RPS, RPS)])
    pltpu.sync_copy(zeros_hbm, acc_sh.at[pl.ds(s * RPS, RPS)])
    plsc.subcore_barrier()

    def body(st, carry):
      for b in range(NBUF):
        ch = st * NBUF + b
        pltpu.make_async_copy(op_sh.at[src_v.at[ch]], bufs[b], sems[b]).wait()
        pltpu.sync_copy(bufs[b], acc_sh.at[dst_v.at[ch]], add=True)
        nxt = ch + NBUF

        @pl.when(nxt < HALF)
        def _():
          pltpu.async_copy(op_sh.at[src_v.at[nxt]], bufs[b], sems[b])

      return carry

    for h in range(NHALF):
      pltpu.sync_copy(src_hbm.at[wid, h], src_v)
      pltpu.sync_copy(dst_hbm.at[wid, h], dst_v)
      for b in range(NBUF):
        pltpu.async_copy(op_sh.at[src_v.at[b]], bufs[b], sems[b])
      lax.fori_loop(0, HALF // NBUF, body, 0)
    plsc.subcore_barrier()
    pltpu.sync_copy(acc_sh.at[pl.ds(s * RPS, RPS)],
                    out_hbm.at[c, pl.ds(s * RPS, RPS)])

  return agg


_agg64 = _make_agg(N_CLS)


@functools.partial(
    pl.kernel,
    out_type=jax.ShapeDtypeStruct((2, NPAD, 16), jnp.float32),
    mesh=_mesh,
    scratch_types=[
        pltpu.VMEM((HALF, CHUNK), jnp.int32),
        pltpu.VMEM((CHUNK, 16), jnp.float32),
        pltpu.VMEM_SHARED((NPAD, 16), jnp.float32),
    ],
    compiler_params=_sc_params,
)
def _deg_kernel(dst_hbm, onehot_hbm, zeros_hbm, out_hbm, dst_v, ones_v, acc_sh):
  c = lax.axis_index("c")
  s = lax.axis_index("s")
  wid = s * 2 + c
  pltpu.sync_copy(zeros_hbm, acc_sh.at[pl.ds(s * RPS, RPS)])
  pltpu.sync_copy(onehot_hbm, ones_v)
  plsc.subcore_barrier()

  def body(ch, carry):
    pltpu.sync_copy(ones_v, acc_sh.at[dst_v.at[ch]], add=True)
    return carry

  for h in range(NHALF):
    pltpu.sync_copy(dst_hbm.at[wid, h], dst_v)
    lax.fori_loop(0, HALF, body, 0)
  plsc.subcore_barrier()
  pltpu.sync_copy(acc_sh.at[pl.ds(s * RPS, RPS)],
                  out_hbm.at[c, pl.ds(s * RPS, RPS)])


def _isd_of(da_ref, db_ref):
  deg = da_ref[:, 0:1] + db_ref[:, 0:1]
  return lax.rsqrt(jnp.maximum(deg, 1.0))


def _mm1_body(x_ref, w_ref, da_ref, db_ref, oa_ref, ob_ref):
  isd = _isd_of(da_ref, db_ref)
  g = jnp.dot(x_ref[...], w_ref[...], preferred_element_type=jnp.float32) * isd
  oa_ref[...] = g[:, :N_CLS]
  ob_ref[...] = g[:, N_CLS:]


def _mm2_body(aa0_ref, aa1_ref, ab0_ref, ab1_ref, da_ref, db_ref,
              b1a_ref, b1b_ref, w2a_ref, w2b_ref, o_ref):
  isd = _isd_of(da_ref, db_ref)
  ha = jnp.maximum(isd * (aa0_ref[...] + aa1_ref[...]) + b1a_ref[...], 0.0)
  hb = jnp.maximum(isd * (ab0_ref[...] + ab1_ref[...]) + b1b_ref[...], 0.0)
  o_ref[...] = (jnp.dot(ha, w2a_ref[...], preferred_element_type=jnp.float32)
                + jnp.dot(hb, w2b_ref[...], preferred_element_type=jnp.float32)
                ) * isd


def _fin_body(aa_ref, ab_ref, da_ref, db_ref, b2_ref, o_ref):
  isd = _isd_of(da_ref, db_ref)
  o_ref[...] = isd * (aa_ref[...] + ab_ref[...]) + b2_ref[...]


_BLK = 512
_GRID = NPAD // _BLK


def _row_spec(d):
  return pl.BlockSpec((_BLK, d), lambda i: (i, 0))


def _full_spec(r, c):
  return pl.BlockSpec((r, c), lambda i: (0, 0))


_mm1 = pl.pallas_call(
    _mm1_body,
    grid=(_GRID,),
    in_specs=[_row_spec(D_IN), _full_spec(D_IN, D_HID),
              _row_spec(16), _row_spec(16)],
    out_specs=[_row_spec(N_CLS), _row_spec(N_CLS)],
    out_shape=[jax.ShapeDtypeStruct((NPAD, N_CLS), jnp.float32),
               jax.ShapeDtypeStruct((NPAD, N_CLS), jnp.float32)],
)

_mm2 = pl.pallas_call(
    _mm2_body,
    grid=(_GRID,),
    in_specs=[_row_spec(N_CLS), _row_spec(N_CLS),
              _row_spec(N_CLS), _row_spec(N_CLS),
              _row_spec(16), _row_spec(16),
              _full_spec(1, N_CLS), _full_spec(1, N_CLS),
              _full_spec(N_CLS, N_CLS), _full_spec(N_CLS, N_CLS)],
    out_specs=_row_spec(N_CLS),
    out_shape=jax.ShapeDtypeStruct((NPAD, N_CLS), jnp.float32),
)

_fin = pl.pallas_call(
    _fin_body,
    grid=(_GRID,),
    in_specs=[_row_spec(N_CLS), _row_spec(N_CLS), _row_spec(16), _row_spec(16),
              _full_spec(1, N_CLS)],
    out_specs=_row_spec(N_CLS),
    out_shape=jax.ShapeDtypeStruct((NPAD, N_CLS), jnp.float32),
)


def kernel(x, edge_index, W1, b1, W2, b2):
  src = edge_index[0].astype(jnp.int32)
  dst = edge_index[1].astype(jnp.int32)
  pad = jnp.full((EPAD - E,), N, jnp.int32)
  src3 = jnp.concatenate([src, pad]).reshape(NW, NHALF, HALF, CHUNK)
  dst3 = jnp.concatenate([dst, pad]).reshape(NW, NHALF, HALF, CHUNK)
  xp = jnp.zeros((NPAD, D_IN), jnp.float32).at[:N].set(x)

  onehot = jnp.zeros((CHUNK, 16), jnp.float32).at[:, 0].set(1.0)
  z16 = jnp.zeros((RPS, 16), jnp.float32)
  z64 = jnp.zeros((RPS, N_CLS), jnp.float32)

  deg = _deg_kernel(dst3, onehot, z16)
  g1a, g1b = _mm1(xp, W1, deg[0], deg[1])
  agg1a = _agg64(g1a, src3, dst3, z64)
  agg1b = _agg64(g1b, src3, dst3, z64)
  g2 = _mm2(agg1a[0], agg1a[1], agg1b[0], agg1b[1], deg[0], deg[1],
            b1[:N_CLS].reshape(1, N_CLS), b1[N_CLS:].reshape(1, N_CLS),
            W2[:N_CLS], W2[N_CLS:])
  agg2 = _agg64(g2, src3, dst3, z64)
  out = _fin(agg2[0], agg2[1], deg[0], deg[1], b2.reshape(1, N_CLS))
  return out[:N]


# speedup vs baseline: 2.1914x; 1.0648x over previous
import functools

import jax
import jax.numpy as jnp
from jax import lax
from jax.experimental import pallas as pl
from jax.experimental.pallas import tpu as pltpu
from jax.experimental.pallas import tpu_sc as plsc

N = 10000
NPAD = 10240
E = 320000
D_IN = 128
D_HID = 128
N_CLS = 64

CHUNK = 128
SPT = 160
NSTAGE = 4
SCH = SPT // NSTAGE
NBUF = 4
EPAD = 16 * SPT * CHUNK
RPS = NPAD // 16

_mesh = plsc.VectorSubcoreMesh(core_axis_name="c", subcore_axis_name="s")
_sc_params = pltpu.CompilerParams(use_tc_tiling_on_sc=False)


def _make_agg(D):
  D2 = D // 2

  @functools.partial(
      pl.kernel,
      out_type=jax.ShapeDtypeStruct((2, NPAD, D2), jnp.float32),
      mesh=_mesh,
      scratch_types=[
          pltpu.VMEM((SCH, CHUNK), jnp.int32),
          pltpu.VMEM((SCH, CHUNK), jnp.int32),
          [pltpu.VMEM((CHUNK, D2), jnp.float32) for _ in range(NBUF)],
          pltpu.VMEM_SHARED((NPAD, D2), jnp.float32),
          pltpu.VMEM_SHARED((NPAD, D2), jnp.float32),
          [pltpu.SemaphoreType.DMA for _ in range(NBUF)],
      ],
      compiler_params=_sc_params,
  )
  def agg(g_hbm, src_hbm, dst_hbm, zeros_hbm, out_hbm,
          src_v, dst_v, bufs, op_sh, acc_sh, sems):
    c = lax.axis_index("c")
    s = lax.axis_index("s")
    pltpu.sync_copy(g_hbm.at[c, pl.ds(s * RPS, RPS)],
                    op_sh.at[pl.ds(s * RPS, RPS)])
    pltpu.sync_copy(zeros_hbm, acc_sh.at[pl.ds(s * RPS, RPS)])
    plsc.subcore_barrier()

    def body(st, carry):
      for b in range(NBUF):
        ch = st * NBUF + b
        pltpu.make_async_copy(op_sh.at[src_v.at[ch]], bufs[b], sems[b]).wait()
        pltpu.sync_copy(bufs[b], acc_sh.at[dst_v.at[ch]], add=True)
        nxt = ch + NBUF

        @pl.when(nxt < SCH)
        def _():
          pltpu.async_copy(op_sh.at[src_v.at[nxt]], bufs[b], sems[b])

      return carry

    for stage in range(NSTAGE):
      pltpu.sync_copy(src_hbm.at[s, stage], src_v)
      pltpu.sync_copy(dst_hbm.at[s, stage], dst_v)
      for b in range(NBUF):
        pltpu.async_copy(op_sh.at[src_v.at[b]], bufs[b], sems[b])
      lax.fori_loop(0, SCH // NBUF, body, 0)
    plsc.subcore_barrier()
    pltpu.sync_copy(acc_sh.at[pl.ds(s * RPS, RPS)],
                    out_hbm.at[c, pl.ds(s * RPS, RPS)])

  return agg


_agg128 = _make_agg(D_HID)
_agg64 = _make_agg(N_CLS)


@functools.partial(
    pl.kernel,
    out_type=jax.ShapeDtypeStruct((2, NPAD, 16), jnp.float32),
    mesh=_mesh,
    scratch_types=[
        pltpu.VMEM((SCH, CHUNK), jnp.int32),
        pltpu.VMEM((CHUNK, 16), jnp.float32),
        pltpu.VMEM_SHARED((NPAD, 16), jnp.float32),
    ],
    compiler_params=_sc_params,
)
def _deg_kernel(dst_hbm, onehot_hbm, zeros_hbm, out_hbm, dst_v, ones_v, acc_sh):
  c = lax.axis_index("c")
  s = lax.axis_index("s")
  pltpu.sync_copy(zeros_hbm, acc_sh.at[pl.ds(s * RPS, RPS)])
  pltpu.sync_copy(onehot_hbm, ones_v)
  plsc.subcore_barrier()

  def body(ch, carry):
    pltpu.sync_copy(ones_v, acc_sh.at[dst_v.at[ch]], add=True)
    return carry

  for k in range(NSTAGE // 2):
    pltpu.sync_copy(dst_hbm.at[s, c + 2 * k], dst_v)
    lax.fori_loop(0, SCH, body, 0)
  plsc.subcore_barrier()
  pltpu.sync_copy(acc_sh.at[pl.ds(s * RPS, RPS)],
                  out_hbm.at[c, pl.ds(s * RPS, RPS)])


def _isd_of(da_ref, db_ref):
  deg = da_ref[:, 0:1] + db_ref[:, 0:1]
  return lax.rsqrt(jnp.maximum(deg, 1.0))


def _mm1_body(x_ref, w_ref, da_ref, db_ref, o_ref):
  isd = _isd_of(da_ref, db_ref)
  g = jnp.dot(x_ref[...], w_ref[...], preferred_element_type=jnp.float32) * isd
  o_ref[0] = g[:, :D_HID // 2]
  o_ref[1] = g[:, D_HID // 2:]


def _mm2_body(a_ref, da_ref, db_ref, b1_ref, w2_ref, o_ref):
  isd = _isd_of(da_ref, db_ref)
  ha = jnp.maximum(isd * a_ref[0] + b1_ref[:, :D_HID // 2], 0.0)
  hb = jnp.maximum(isd * a_ref[1] + b1_ref[:, D_HID // 2:], 0.0)
  g = (jnp.dot(ha, w2_ref[:D_HID // 2], preferred_element_type=jnp.float32)
       + jnp.dot(hb, w2_ref[D_HID // 2:], preferred_element_type=jnp.float32)
       ) * isd
  o_ref[0] = g[:, :N_CLS // 2]
  o_ref[1] = g[:, N_CLS // 2:]


def _fin_body(a_ref, da_ref, db_ref, b2_ref, o_ref):
  isd = _isd_of(da_ref, db_ref)
  o_ref[:, :N_CLS // 2] = isd * a_ref[0] + b2_ref[:, :N_CLS // 2]
  o_ref[:, N_CLS // 2:] = isd * a_ref[1] + b2_ref[:, N_CLS // 2:]


_BLK = 512
_GRID = NPAD // _BLK


def _row_spec(d):
  return pl.BlockSpec((_BLK, d), lambda i: (i, 0))


def _split_spec(d):
  return pl.BlockSpec((2, _BLK, d), lambda i: (0, i, 0))


def _full_spec(r, c):
  return pl.BlockSpec((r, c), lambda i: (0, 0))


_mm1 = pl.pallas_call(
    _mm1_body,
    grid=(_GRID,),
    in_specs=[_row_spec(D_IN), _full_spec(D_IN, D_HID),
              _row_spec(16), _row_spec(16)],
    out_specs=_split_spec(D_HID // 2),
    out_shape=jax.ShapeDtypeStruct((2, NPAD, D_HID // 2), jnp.float32),
)

_mm2 = pl.pallas_call(
    _mm2_body,
    grid=(_GRID,),
    in_specs=[_split_spec(D_HID // 2), _row_spec(16), _row_spec(16),
              _full_spec(1, D_HID), _full_spec(D_HID, N_CLS)],
    out_specs=_split_spec(N_CLS // 2),
    out_shape=jax.ShapeDtypeStruct((2, NPAD, N_CLS // 2), jnp.float32),
)

_fin = pl.pallas_call(
    _fin_body,
    grid=(_GRID,),
    in_specs=[_split_spec(N_CLS // 2), _row_spec(16), _row_spec(16),
              _full_spec(1, N_CLS)],
    out_specs=_row_spec(N_CLS),
    out_shape=jax.ShapeDtypeStruct((NPAD, N_CLS), jnp.float32),
)


def kernel(x, edge_index, W1, b1, W2, b2):
  src = edge_index[0].astype(jnp.int32)
  dst = edge_index[1].astype(jnp.int32)
  pad = jnp.full((EPAD - E,), N, jnp.int32)
  src4 = jnp.concatenate([src, pad]).reshape(16, NSTAGE, SCH, CHUNK)
  dst4 = jnp.concatenate([dst, pad]).reshape(16, NSTAGE, SCH, CHUNK)
  xp = jnp.zeros((NPAD, D_IN), jnp.float32).at[:N].set(x)

  onehot = jnp.zeros((CHUNK, 16), jnp.float32).at[:, 0].set(1.0)
  z16 = jnp.zeros((RPS, 16), jnp.float32)
  z64 = jnp.zeros((RPS, D_HID // 2), jnp.float32)
  z32 = jnp.zeros((RPS, N_CLS // 2), jnp.float32)

  deg = _deg_kernel(dst4, onehot, z16)
  g1 = _mm1(xp, W1, deg[0], deg[1])
  agg1 = _agg128(g1, src4, dst4, z64)
  g2 = _mm2(agg1, deg[0], deg[1], b1.reshape(1, D_HID), W2)
  agg2 = _agg64(g2, src4, dst4, z32)
  out = _fin(agg2, deg[0], deg[1], b2.reshape(1, N_CLS))
  return out[:N]


# speedup vs baseline: 2.4213x; 1.1049x over previous
import functools

import jax
import jax.numpy as jnp
from jax import lax
from jax.experimental import pallas as pl
from jax.experimental.pallas import tpu as pltpu
from jax.experimental.pallas import tpu_sc as plsc

N = 10000
NPAD = 10240
E = 320000
D_IN = 128
D_HID = 128
N_CLS = 64

CHUNK = 64
SPT = 320
NSTAGE = 4
SCH = SPT // NSTAGE
NBUF = 8
LAG = 4
EPAD = 16 * SPT * CHUNK
RPS = NPAD // 16

_mesh = plsc.VectorSubcoreMesh(core_axis_name="c", subcore_axis_name="s")
_sc_params = pltpu.CompilerParams(use_tc_tiling_on_sc=False)


def _make_agg(D):
  D2 = D // 2

  @functools.partial(
      pl.kernel,
      out_type=jax.ShapeDtypeStruct((2, NPAD, D2), jnp.float32),
      mesh=_mesh,
      scratch_types=[
          pltpu.VMEM((SCH, CHUNK), jnp.int32),
          pltpu.VMEM((SCH, CHUNK), jnp.int32),
          [pltpu.VMEM((CHUNK, D2), jnp.float32) for _ in range(NBUF)],
          pltpu.VMEM_SHARED((NPAD, D2), jnp.float32),
          pltpu.VMEM_SHARED((NPAD, D2), jnp.float32),
          [pltpu.SemaphoreType.DMA for _ in range(NBUF)],
          [pltpu.SemaphoreType.DMA for _ in range(NBUF)],
      ],
      compiler_params=_sc_params,
  )
  def agg(g_hbm, src_hbm, dst_hbm, zeros_hbm, out_hbm,
          src_v, dst_v, bufs, op_sh, acc_sh, gsems, ssems):
    c = lax.axis_index("c")
    s = lax.axis_index("s")
    pltpu.sync_copy(g_hbm.at[c, pl.ds(s * RPS, RPS)],
                    op_sh.at[pl.ds(s * RPS, RPS)])
    pltpu.sync_copy(zeros_hbm, acc_sh.at[pl.ds(s * RPS, RPS)])
    plsc.subcore_barrier()

    def body(st, carry):
      for j in range(NBUF):
        ch = st * NBUF + j
        b = j
        pltpu.make_async_copy(op_sh.at[src_v.at[ch]], bufs[b], gsems[b]).wait()
        pltpu.async_copy(bufs[b], acc_sh.at[dst_v.at[ch]], ssems[b], add=True)
        b2 = (j + LAG) % NBUF
        nxt = ch + LAG

        @pl.when(jnp.logical_and(nxt >= NBUF, nxt < SCH))
        def _():
          pltpu.make_async_copy(
              bufs[b2], acc_sh.at[dst_v.at[ch]], ssems[b2]).wait()

        @pl.when(nxt < SCH)
        def _():
          pltpu.async_copy(op_sh.at[src_v.at[nxt]], bufs[b2], gsems[b2])

      return carry

    for stage in range(NSTAGE):
      pltpu.sync_copy(src_hbm.at[s, stage], src_v)
      pltpu.sync_copy(dst_hbm.at[s, stage], dst_v)
      for b in range(LAG):
        pltpu.async_copy(op_sh.at[src_v.at[b]], bufs[b], gsems[b])
      lax.fori_loop(0, SCH // NBUF, body, 0)
      for b in range(NBUF):
        pltpu.make_async_copy(bufs[b], acc_sh.at[dst_v.at[b]], ssems[b]).wait()
    plsc.subcore_barrier()
    pltpu.sync_copy(acc_sh.at[pl.ds(s * RPS, RPS)],
                    out_hbm.at[c, pl.ds(s * RPS, RPS)])

  return agg


_agg128 = _make_agg(D_HID)
_agg64 = _make_agg(N_CLS)


@functools.partial(
    pl.kernel,
    out_type=jax.ShapeDtypeStruct((2, NPAD, 16), jnp.float32),
    mesh=_mesh,
    scratch_types=[
        pltpu.VMEM((SCH, CHUNK), jnp.int32),
        pltpu.VMEM((CHUNK, 16), jnp.float32),
        pltpu.VMEM_SHARED((NPAD, 16), jnp.float32),
    ],
    compiler_params=_sc_params,
)
def _deg_kernel(dst_hbm, onehot_hbm, zeros_hbm, out_hbm, dst_v, ones_v, acc_sh):
  c = lax.axis_index("c")
  s = lax.axis_index("s")
  pltpu.sync_copy(zeros_hbm, acc_sh.at[pl.ds(s * RPS, RPS)])
  pltpu.sync_copy(onehot_hbm, ones_v)
  plsc.subcore_barrier()

  def body(ch, carry):
    pltpu.sync_copy(ones_v, acc_sh.at[dst_v.at[ch]], add=True)
    return carry

  for k in range(NSTAGE // 2):
    pltpu.sync_copy(dst_hbm.at[s, c + 2 * k], dst_v)
    lax.fori_loop(0, SCH, body, 0)
  plsc.subcore_barrier()
  pltpu.sync_copy(acc_sh.at[pl.ds(s * RPS, RPS)],
                  out_hbm.at[c, pl.ds(s * RPS, RPS)])


def _isd_of(da_ref, db_ref):
  deg = da_ref[:, 0:1] + db_ref[:, 0:1]
  return lax.rsqrt(jnp.maximum(deg, 1.0))


def _mm1_body(x_ref, w_ref, da_ref, db_ref, o_ref):
  isd = _isd_of(da_ref, db_ref)
  g = jnp.dot(x_ref[...], w_ref[...], preferred_element_type=jnp.float32) * isd
  o_ref[0] = g[:, :D_HID // 2]
  o_ref[1] = g[:, D_HID // 2:]


def _mm2_body(a_ref, da_ref, db_ref, b1_ref, w2_ref, o_ref):
  isd = _isd_of(da_ref, db_ref)
  ha = jnp.maximum(isd * a_ref[0] + b1_ref[:, :D_HID // 2], 0.0)
  hb = jnp.maximum(isd * a_ref[1] + b1_ref[:, D_HID // 2:], 0.0)
  g = (jnp.dot(ha, w2_ref[:D_HID // 2], preferred_element_type=jnp.float32)
       + jnp.dot(hb, w2_ref[D_HID // 2:], preferred_element_type=jnp.float32)
       ) * isd
  o_ref[0] = g[:, :N_CLS // 2]
  o_ref[1] = g[:, N_CLS // 2:]


def _fin_body(a_ref, da_ref, db_ref, b2_ref, o_ref):
  isd = _isd_of(da_ref, db_ref)
  o_ref[:, :N_CLS // 2] = isd * a_ref[0] + b2_ref[:, :N_CLS // 2]
  o_ref[:, N_CLS // 2:] = isd * a_ref[1] + b2_ref[:, N_CLS // 2:]


_BLK = 512
_GRID = NPAD // _BLK


def _row_spec(d):
  return pl.BlockSpec((_BLK, d), lambda i: (i, 0))


def _split_spec(d):
  return pl.BlockSpec((2, _BLK, d), lambda i: (0, i, 0))


def _full_spec(r, c):
  return pl.BlockSpec((r, c), lambda i: (0, 0))


_mm1 = pl.pallas_call(
    _mm1_body,
    grid=(_GRID,),
    in_specs=[_row_spec(D_IN), _full_spec(D_IN, D_HID),
              _row_spec(16), _row_spec(16)],
    out_specs=_split_spec(D_HID // 2),
    out_shape=jax.ShapeDtypeStruct((2, NPAD, D_HID // 2), jnp.float32),
)

_mm2 = pl.pallas_call(
    _mm2_body,
    grid=(_GRID,),
    in_specs=[_split_spec(D_HID // 2), _row_spec(16), _row_spec(16),
              _full_spec(1, D_HID), _full_spec(D_HID, N_CLS)],
    out_specs=_split_spec(N_CLS // 2),
    out_shape=jax.ShapeDtypeStruct((2, NPAD, N_CLS // 2), jnp.float32),
)

_fin = pl.pallas_call(
    _fin_body,
    grid=(_GRID,),
    in_specs=[_split_spec(N_CLS // 2), _row_spec(16), _row_spec(16),
              _full_spec(1, N_CLS)],
    out_specs=_row_spec(N_CLS),
    out_shape=jax.ShapeDtypeStruct((NPAD, N_CLS), jnp.float32),
)


def kernel(x, edge_index, W1, b1, W2, b2):
  src = edge_index[0].astype(jnp.int32)
  dst = edge_index[1].astype(jnp.int32)
  pad = jnp.full((EPAD - E,), N, jnp.int32)
  src4 = jnp.concatenate([src, pad]).reshape(16, NSTAGE, SCH, CHUNK)
  dst4 = jnp.concatenate([dst, pad]).reshape(16, NSTAGE, SCH, CHUNK)
  xp = jnp.zeros((NPAD, D_IN), jnp.float32).at[:N].set(x)

  onehot = jnp.zeros((CHUNK, 16), jnp.float32).at[:, 0].set(1.0)
  z16 = jnp.zeros((RPS, 16), jnp.float32)
  z64 = jnp.zeros((RPS, D_HID // 2), jnp.float32)
  z32 = jnp.zeros((RPS, N_CLS // 2), jnp.float32)

  deg = _deg_kernel(dst4, onehot, z16)
  g1 = _mm1(xp, W1, deg[0], deg[1])
  agg1 = _agg128(g1, src4, dst4, z64)
  g2 = _mm2(agg1, deg[0], deg[1], b1.reshape(1, D_HID), W2)
  agg2 = _agg64(g2, src4, dst4, z32)
  out = _fin(agg2, deg[0], deg[1], b2.reshape(1, N_CLS))
  return out[:N]


# speedup vs baseline: 2.4614x; 1.0166x over previous
import functools

import jax
import jax.numpy as jnp
from jax import lax
from jax.experimental import pallas as pl
from jax.experimental.pallas import tpu as pltpu
from jax.experimental.pallas import tpu_sc as plsc

N = 10000
NPAD = 10240
E = 320000
D_IN = 128
D_HID = 128
N_CLS = 64

CHUNK = 64
SPT = 320
NSTAGE = 4
SCH = SPT // NSTAGE
NBUF = 8
LAG = 4
EPAD = 16 * SPT * CHUNK
RPS = NPAD // 16
OPS = N // 16

_mesh = plsc.VectorSubcoreMesh(core_axis_name="c", subcore_axis_name="s")
_sc_params = pltpu.CompilerParams(use_tc_tiling_on_sc=False)


def _make_agg(D):
  D2 = D // 2

  @functools.partial(
      pl.kernel,
      out_type=jax.ShapeDtypeStruct((2, NPAD, D2), jnp.float32),
      mesh=_mesh,
      scratch_types=[
          pltpu.VMEM((SCH, CHUNK), jnp.int32),
          pltpu.VMEM((SCH, CHUNK), jnp.int32),
          [pltpu.VMEM((CHUNK, D2), jnp.float32) for _ in range(NBUF)],
          pltpu.VMEM_SHARED((NPAD, D2), jnp.float32),
          pltpu.VMEM_SHARED((NPAD, D2), jnp.float32),
          [pltpu.SemaphoreType.DMA for _ in range(NBUF)],
          [pltpu.SemaphoreType.DMA for _ in range(NBUF)],
      ],
      compiler_params=_sc_params,
  )
  def agg(g_hbm, src_hbm, dst_hbm, zeros_hbm, out_hbm,
          src_v, dst_v, bufs, op_sh, acc_sh, gsems, ssems):
    c = lax.axis_index("c")
    s = lax.axis_index("s")
    pltpu.sync_copy(g_hbm.at[c, pl.ds(s * OPS, OPS)],
                    op_sh.at[pl.ds(s * OPS, OPS)])
    pltpu.sync_copy(zeros_hbm, acc_sh.at[pl.ds(s * RPS, RPS)])
    plsc.subcore_barrier()

    def body(st, carry):
      for j in range(NBUF):
        ch = st * NBUF + j
        b = j
        pltpu.make_async_copy(op_sh.at[src_v.at[ch]], bufs[b], gsems[b]).wait()
        pltpu.async_copy(bufs[b], acc_sh.at[dst_v.at[ch]], ssems[b], add=True)
        b2 = (j + LAG) % NBUF
        nxt = ch + LAG

        @pl.when(jnp.logical_and(nxt >= NBUF, nxt < SCH))
        def _():
          pltpu.make_async_copy(
              bufs[b2], acc_sh.at[dst_v.at[ch]], ssems[b2]).wait()

        @pl.when(nxt < SCH)
        def _():
          pltpu.async_copy(op_sh.at[src_v.at[nxt]], bufs[b2], gsems[b2])

      return carry

    for stage in range(NSTAGE):
      pltpu.sync_copy(src_hbm.at[s, stage], src_v)
      pltpu.sync_copy(dst_hbm.at[s, stage], dst_v)
      for b in range(LAG):
        pltpu.async_copy(op_sh.at[src_v.at[b]], bufs[b], gsems[b])
      lax.fori_loop(0, SCH // NBUF, body, 0)
      for b in range(NBUF):
        pltpu.make_async_copy(bufs[b], acc_sh.at[dst_v.at[b]], ssems[b]).wait()
    plsc.subcore_barrier()
    pltpu.sync_copy(acc_sh.at[pl.ds(s * RPS, RPS)],
                    out_hbm.at[c, pl.ds(s * RPS, RPS)])

  return agg


_agg128 = _make_agg(D_HID)
_agg64 = _make_agg(N_CLS)


@functools.partial(
    pl.kernel,
    out_type=jax.ShapeDtypeStruct((2, NPAD, 16), jnp.float32),
    mesh=_mesh,
    scratch_types=[
        pltpu.VMEM((SCH, CHUNK), jnp.int32),
        pltpu.VMEM((CHUNK, 16), jnp.float32),
        pltpu.VMEM_SHARED((NPAD, 16), jnp.float32),
    ],
    compiler_params=_sc_params,
)
def _deg_kernel(dst_hbm, onehot_hbm, zeros_hbm, out_hbm, dst_v, ones_v, acc_sh):
  c = lax.axis_index("c")
  s = lax.axis_index("s")
  pltpu.sync_copy(zeros_hbm, acc_sh.at[pl.ds(s * RPS, RPS)])
  pltpu.sync_copy(onehot_hbm, ones_v)
  plsc.subcore_barrier()

  def body(ch, carry):
    pltpu.sync_copy(ones_v, acc_sh.at[dst_v.at[ch]], add=True)
    return carry

  for k in range(NSTAGE // 2):
    pltpu.sync_copy(dst_hbm.at[s, c + 2 * k], dst_v)
    lax.fori_loop(0, SCH, body, 0)
  plsc.subcore_barrier()
  pltpu.sync_copy(acc_sh.at[pl.ds(s * RPS, RPS)],
                  out_hbm.at[c, pl.ds(s * RPS, RPS)])


def _mm1_body(x_ref, w_ref, o_ref):
  o_ref[...] = jnp.dot(x_ref[...], w_ref[...],
                       preferred_element_type=jnp.float32)


def _mm2_body(h_ref, w_ref, o_ref):
  o_ref[...] = jnp.dot(h_ref[...], w_ref[...],
                       preferred_element_type=jnp.float32)


_BLK = 400
_GRID = N // _BLK


def _row_spec(d):
  return pl.BlockSpec((_BLK, d), lambda i: (i, 0))


def _full_spec(r, c):
  return pl.BlockSpec((r, c), lambda i: (0, 0))


_mm1 = pl.pallas_call(
    _mm1_body,
    grid=(_GRID,),
    in_specs=[_row_spec(D_IN), _full_spec(D_IN, D_HID)],
    out_specs=_row_spec(D_HID),
    out_shape=jax.ShapeDtypeStruct((N, D_HID), jnp.float32),
)

_mm2 = pl.pallas_call(
    _mm2_body,
    grid=(_GRID,),
    in_specs=[_row_spec(D_HID), _full_spec(D_HID, N_CLS)],
    out_specs=_row_spec(N_CLS),
    out_shape=jax.ShapeDtypeStruct((N, N_CLS), jnp.float32),
)


def kernel(x, edge_index, W1, b1, W2, b2):
  src = edge_index[0].astype(jnp.int32)
  dst = edge_index[1].astype(jnp.int32)
  src4 = jnp.concatenate([src, jnp.zeros((EPAD - E,), jnp.int32)])
  dst4 = jnp.concatenate([dst, jnp.full((EPAD - E,), NPAD - 1, jnp.int32)])
  src4 = src4.reshape(16, NSTAGE, SCH, CHUNK)
  dst4 = dst4.reshape(16, NSTAGE, SCH, CHUNK)

  onehot = jnp.zeros((CHUNK, 16), jnp.float32).at[:, 0].set(1.0)
  z16 = jnp.zeros((RPS, 16), jnp.float32)
  z64 = jnp.zeros((RPS, D_HID // 2), jnp.float32)
  z32 = jnp.zeros((RPS, N_CLS // 2), jnp.float32)

  deg = _deg_kernel(dst4, onehot, z16)
  raw1 = _mm1(x, W1)

  isd = lax.rsqrt(jnp.maximum(deg[0, :N, 0:1] + deg[1, :N, 0:1], 1.0))
  g1 = jnp.stack([raw1[:, :D_HID // 2] * isd, raw1[:, D_HID // 2:] * isd])
  agg1 = _agg128(g1, src4, dst4, z64)

  h1 = jnp.maximum(
      isd * jnp.concatenate([agg1[0, :N], agg1[1, :N]], axis=1) + b1, 0.0)
  raw2 = _mm2(h1, W2)
  g2 = jnp.stack([raw2[:, :N_CLS // 2] * isd, raw2[:, N_CLS // 2:] * isd])
  agg2 = _agg64(g2, src4, dst4, z32)

  return isd * jnp.concatenate([agg2[0, :N], agg2[1, :N]], axis=1) + b2


# speedup vs baseline: 2.4752x; 1.0056x over previous
import functools

import jax
import jax.numpy as jnp
from jax import lax
from jax.experimental import pallas as pl
from jax.experimental.pallas import tpu as pltpu
from jax.experimental.pallas import tpu_sc as plsc

N = 10000
NPAD = 10240
E = 320000
D_IN = 128
D_HID = 128
N_CLS = 64

CHUNK = 128
SPT = 160
NSTAGE = 4
SCH = SPT // NSTAGE
NBUF = 4
LAG = 2
EPAD = 16 * SPT * CHUNK
RPS = NPAD // 16
OPS = N // 16

_mesh = plsc.VectorSubcoreMesh(core_axis_name="c", subcore_axis_name="s")
_sc_params = pltpu.CompilerParams(use_tc_tiling_on_sc=False)


def _make_agg(D):
  D2 = D // 2

  @functools.partial(
      pl.kernel,
      out_type=jax.ShapeDtypeStruct((2, NPAD, D2), jnp.float32),
      mesh=_mesh,
      scratch_types=[
          pltpu.VMEM((SCH, CHUNK), jnp.int32),
          pltpu.VMEM((SCH, CHUNK), jnp.int32),
          [pltpu.VMEM((CHUNK, D2), jnp.float32) for _ in range(NBUF)],
          pltpu.VMEM_SHARED((NPAD, D2), jnp.float32),
          pltpu.VMEM_SHARED((NPAD, D2), jnp.float32),
          [pltpu.SemaphoreType.DMA for _ in range(NBUF)],
          [pltpu.SemaphoreType.DMA for _ in range(NBUF)],
      ],
      compiler_params=_sc_params,
  )
  def agg(g_hbm, src_hbm, dst_hbm, zeros_hbm, out_hbm,
          src_v, dst_v, bufs, op_sh, acc_sh, gsems, ssems):
    c = lax.axis_index("c")
    s = lax.axis_index("s")
    pltpu.sync_copy(g_hbm.at[c, pl.ds(s * OPS, OPS)],
                    op_sh.at[pl.ds(s * OPS, OPS)])
    pltpu.sync_copy(zeros_hbm, acc_sh.at[pl.ds(s * RPS, RPS)])
    plsc.subcore_barrier()

    def body(st, carry):
      for j in range(NBUF):
        ch = st * NBUF + j
        b = j
        pltpu.make_async_copy(op_sh.at[src_v.at[ch]], bufs[b], gsems[b]).wait()
        pltpu.async_copy(bufs[b], acc_sh.at[dst_v.at[ch]], ssems[b], add=True)
        b2 = (j + LAG) % NBUF
        nxt = ch + LAG

        @pl.when(jnp.logical_and(nxt >= NBUF, nxt < SCH))
        def _():
          pltpu.make_async_copy(
              bufs[b2], acc_sh.at[dst_v.at[ch]], ssems[b2]).wait()

        @pl.when(nxt < SCH)
        def _():
          pltpu.async_copy(op_sh.at[src_v.at[nxt]], bufs[b2], gsems[b2])

      return carry

    for stage in range(NSTAGE):
      pltpu.sync_copy(src_hbm.at[s, stage], src_v)
      pltpu.sync_copy(dst_hbm.at[s, stage], dst_v)
      for b in range(LAG):
        pltpu.async_copy(op_sh.at[src_v.at[b]], bufs[b], gsems[b])
      lax.fori_loop(0, SCH // NBUF, body, 0)
      for b in range(NBUF):
        pltpu.make_async_copy(bufs[b], acc_sh.at[dst_v.at[b]], ssems[b]).wait()
    plsc.subcore_barrier()
    pltpu.sync_copy(acc_sh.at[pl.ds(s * RPS, RPS)],
                    out_hbm.at[c, pl.ds(s * RPS, RPS)])

  return agg


_agg128 = _make_agg(D_HID)
_agg64 = _make_agg(N_CLS)


@functools.partial(
    pl.kernel,
    out_type=jax.ShapeDtypeStruct((2, NPAD, 16), jnp.float32),
    mesh=_mesh,
    scratch_types=[
        pltpu.VMEM((SCH, CHUNK), jnp.int32),
        pltpu.VMEM((CHUNK, 16), jnp.float32),
        pltpu.VMEM_SHARED((NPAD, 16), jnp.float32),
    ],
    compiler_params=_sc_params,
)
def _deg_kernel(dst_hbm, onehot_hbm, zeros_hbm, out_hbm, dst_v, ones_v, acc_sh):
  c = lax.axis_index("c")
  s = lax.axis_index("s")
  pltpu.sync_copy(zeros_hbm, acc_sh.at[pl.ds(s * RPS, RPS)])
  pltpu.sync_copy(onehot_hbm, ones_v)
  plsc.subcore_barrier()

  def body(ch, carry):
    pltpu.sync_copy(ones_v, acc_sh.at[dst_v.at[ch]], add=True)
    return carry

  for k in range(NSTAGE // 2):
    pltpu.sync_copy(dst_hbm.at[s, c + 2 * k], dst_v)
    lax.fori_loop(0, SCH, body, 0)
  plsc.subcore_barrier()
  pltpu.sync_copy(acc_sh.at[pl.ds(s * RPS, RPS)],
                  out_hbm.at[c, pl.ds(s * RPS, RPS)])


def _mm1_body(x_ref, w_ref, o_ref):
  o_ref[...] = jnp.dot(x_ref[...], w_ref[...],
                       preferred_element_type=jnp.float32)


def _mm2_body(h_ref, w_ref, o_ref):
  o_ref[...] = jnp.dot(h_ref[...], w_ref[...],
                       preferred_element_type=jnp.float32)


_BLK = 400
_GRID = N // _BLK


def _row_spec(d):
  return pl.BlockSpec((_BLK, d), lambda i: (i, 0))


def _full_spec(r, c):
  return pl.BlockSpec((r, c), lambda i: (0, 0))


_mm1 = pl.pallas_call(
    _mm1_body,
    grid=(_GRID,),
    in_specs=[_row_spec(D_IN), _full_spec(D_IN, D_HID)],
    out_specs=_row_spec(D_HID),
    out_shape=jax.ShapeDtypeStruct((N, D_HID), jnp.float32),
)

_mm2 = pl.pallas_call(
    _mm2_body,
    grid=(_GRID,),
    in_specs=[_row_spec(D_HID), _full_spec(D_HID, N_CLS)],
    out_specs=_row_spec(N_CLS),
    out_shape=jax.ShapeDtypeStruct((N, N_CLS), jnp.float32),
)


def kernel(x, edge_index, W1, b1, W2, b2):
  src = edge_index[0].astype(jnp.int32)
  dst = edge_index[1].astype(jnp.int32)
  src4 = jnp.concatenate([src, jnp.zeros((EPAD - E,), jnp.int32)])
  dst4 = jnp.concatenate([dst, jnp.full((EPAD - E,), NPAD - 1, jnp.int32)])
  src4 = src4.reshape(16, NSTAGE, SCH, CHUNK)
  dst4 = dst4.reshape(16, NSTAGE, SCH, CHUNK)

  onehot = jnp.zeros((CHUNK, 16), jnp.float32).at[:, 0].set(1.0)
  z16 = jnp.zeros((RPS, 16), jnp.float32)
  z64 = jnp.zeros((RPS, D_HID // 2), jnp.float32)
  z32 = jnp.zeros((RPS, N_CLS // 2), jnp.float32)

  deg = _deg_kernel(dst4, onehot, z16)
  raw1 = _mm1(x, W1)

  isd = lax.rsqrt(jnp.maximum(deg[0, :N, 0:1] + deg[1, :N, 0:1], 1.0))
  g1 = jnp.stack([raw1[:, :D_HID // 2] * isd, raw1[:, D_HID // 2:] * isd])
  agg1 = _agg128(g1, src4, dst4, z64)

  h1 = jnp.maximum(
      isd * jnp.concatenate([agg1[0, :N], agg1[1, :N]], axis=1) + b1, 0.0)
  raw2 = _mm2(h1, W2)
  g2 = jnp.stack([raw2[:, :N_CLS // 2] * isd, raw2[:, N_CLS // 2:] * isd])
  agg2 = _agg64(g2, src4, dst4, z32)

  return isd * jnp.concatenate([agg2[0, :N], agg2[1, :N]], axis=1) + b2


# speedup vs baseline: 2.7854x; 1.1253x over previous
import functools

import jax
import jax.numpy as jnp
from jax import lax
from jax.experimental import pallas as pl
from jax.experimental.pallas import tpu as pltpu
from jax.experimental.pallas import tpu_sc as plsc

N = 10000
NPAD = 10240
E = 320000
D_IN = 128
D_HID = 128
N_CLS = 64

CHUNK = 128
SPT = 160
NSTAGE = 4
SCH = SPT // NSTAGE
NBUF = 4
LAG = 2
EPAD = 16 * SPT * CHUNK
RPS = NPAD // 16

_mesh = plsc.VectorSubcoreMesh(core_axis_name="c", subcore_axis_name="s")
_sc_params = pltpu.CompilerParams(use_tc_tiling_on_sc=False)


def _make_agg(D):
  D2 = D // 2

  @functools.partial(
      pl.kernel,
      out_type=jax.ShapeDtypeStruct((NPAD, D), jnp.float32),
      mesh=_mesh,
      scratch_types=[
          pltpu.VMEM((SCH, CHUNK), jnp.int32),
          pltpu.VMEM((SCH, CHUNK), jnp.int32),
          [pltpu.VMEM((CHUNK, D2), jnp.float32) for _ in range(NBUF)],
          pltpu.VMEM_SHARED((NPAD, D2), jnp.float32),
          pltpu.VMEM_SHARED((NPAD, D2), jnp.float32),
          [pltpu.SemaphoreType.DMA for _ in range(NBUF)],
          [pltpu.SemaphoreType.DMA for _ in range(NBUF)],
      ],
      compiler_params=_sc_params,
  )
  def agg(g_hbm, src_hbm, dst_hbm, zeros_hbm, out_hbm,
          src_v, dst_v, bufs, op_sh, acc_sh, gsems, ssems):
    c = lax.axis_index("c")
    s = lax.axis_index("s")
    pltpu.sync_copy(g_hbm.at[pl.ds(s * RPS, RPS), pl.ds(c * D2, D2)],
                    op_sh.at[pl.ds(s * RPS, RPS)])
    pltpu.sync_copy(zeros_hbm, acc_sh.at[pl.ds(s * RPS, RPS)])
    plsc.subcore_barrier()

    def body(st, carry):
      for j in range(NBUF):
        ch = st * NBUF + j
        b = j
        pltpu.make_async_copy(op_sh.at[src_v.at[ch]], bufs[b], gsems[b]).wait()
        pltpu.async_copy(bufs[b], acc_sh.at[dst_v.at[ch]], ssems[b], add=True)
        b2 = (j + LAG) % NBUF
        nxt = ch + LAG

        @pl.when(jnp.logical_and(nxt >= NBUF, nxt < SCH))
        def _():
          pltpu.make_async_copy(
              bufs[b2], acc_sh.at[dst_v.at[ch]], ssems[b2]).wait()

        @pl.when(nxt < SCH)
        def _():
          pltpu.async_copy(op_sh.at[src_v.at[nxt]], bufs[b2], gsems[b2])

      return carry

    for stage in range(NSTAGE):
      pltpu.sync_copy(src_hbm.at[s, stage], src_v)
      pltpu.sync_copy(dst_hbm.at[s, stage], dst_v)
      for b in range(LAG):
        pltpu.async_copy(op_sh.at[src_v.at[b]], bufs[b], gsems[b])
      lax.fori_loop(0, SCH // NBUF, body, 0)
      for b in range(NBUF):
        pltpu.make_async_copy(bufs[b], acc_sh.at[dst_v.at[b]], ssems[b]).wait()
    plsc.subcore_barrier()
    pltpu.sync_copy(acc_sh.at[pl.ds(s * RPS, RPS)],
                    out_hbm.at[pl.ds(s * RPS, RPS), pl.ds(c * D2, D2)])

  return agg


_agg128 = _make_agg(D_HID)
_agg64 = _make_agg(N_CLS)


@functools.partial(
    pl.kernel,
    out_type=jax.ShapeDtypeStruct((2, NPAD, 16), jnp.float32),
    mesh=_mesh,
    scratch_types=[
        pltpu.VMEM((SCH, CHUNK), jnp.int32),
        pltpu.VMEM((CHUNK, 16), jnp.float32),
        pltpu.VMEM_SHARED((NPAD, 16), jnp.float32),
    ],
    compiler_params=_sc_params,
)
def _deg_kernel(dst_hbm, onehot_hbm, zeros_hbm, out_hbm, dst_v, ones_v, acc_sh):
  c = lax.axis_index("c")
  s = lax.axis_index("s")
  pltpu.sync_copy(zeros_hbm, acc_sh.at[pl.ds(s * RPS, RPS)])
  pltpu.sync_copy(onehot_hbm, ones_v)
  plsc.subcore_barrier()

  def body(ch, carry):
    pltpu.sync_copy(ones_v, acc_sh.at[dst_v.at[ch]], add=True)
    return carry

  for k in range(NSTAGE // 2):
    pltpu.sync_copy(dst_hbm.at[s, c + 2 * k], dst_v)
    lax.fori_loop(0, SCH, body, 0)
  plsc.subcore_barrier()
  pltpu.sync_copy(acc_sh.at[pl.ds(s * RPS, RPS)],
                  out_hbm.at[c, pl.ds(s * RPS, RPS)])


def _mm1_body(x_ref, w_ref, o_ref):
  o_ref[...] = jnp.dot(x_ref[...], w_ref[...],
                       preferred_element_type=jnp.float32)


def _mm2_body(h_ref, w_ref, o_ref):
  o_ref[...] = jnp.dot(h_ref[...], w_ref[...],
                       preferred_element_type=jnp.float32)


_BLK = 512
_GRID = NPAD // _BLK


def _row_spec(d):
  return pl.BlockSpec((_BLK, d), lambda i: (i, 0))


def _full_spec(r, c):
  return pl.BlockSpec((r, c), lambda i: (0, 0))


_mm1 = pl.pallas_call(
    _mm1_body,
    grid=(_GRID,),
    in_specs=[_row_spec(D_IN), _full_spec(D_IN, D_HID)],
    out_specs=_row_spec(D_HID),
    out_shape=jax.ShapeDtypeStruct((NPAD, D_HID), jnp.float32),
)

_mm2 = pl.pallas_call(
    _mm2_body,
    grid=(_GRID,),
    in_specs=[_row_spec(D_HID), _full_spec(D_HID, N_CLS)],
    out_specs=_row_spec(N_CLS),
    out_shape=jax.ShapeDtypeStruct((NPAD, N_CLS), jnp.float32),
)


def kernel(x, edge_index, W1, b1, W2, b2):
  src = edge_index[0].astype(jnp.int32)
  dst = edge_index[1].astype(jnp.int32)
  src4 = jnp.concatenate([src, jnp.zeros((EPAD - E,), jnp.int32)])
  dst4 = jnp.concatenate([dst, jnp.full((EPAD - E,), NPAD - 1, jnp.int32)])
  src4 = src4.reshape(16, NSTAGE, SCH, CHUNK)
  dst4 = dst4.reshape(16, NSTAGE, SCH, CHUNK)
  xp = jnp.zeros((NPAD, D_IN), jnp.float32).at[:N].set(x)

  onehot = jnp.zeros((CHUNK, 16), jnp.float32).at[:, 0].set(1.0)
  z16 = jnp.zeros((RPS, 16), jnp.float32)
  z64 = jnp.zeros((RPS, D_HID // 2), jnp.float32)
  z32 = jnp.zeros((RPS, N_CLS // 2), jnp.float32)

  deg = _deg_kernel(dst4, onehot, z16)
  raw1 = _mm1(xp, W1)

  isd = lax.rsqrt(jnp.maximum(deg[0, :, 0:1] + deg[1, :, 0:1], 1.0))
  g1 = raw1 * isd
  agg1 = _agg128(g1, src4, dst4, z64)

  h1 = jnp.maximum(isd * agg1 + b1, 0.0)
  raw2 = _mm2(h1, W2)
  g2 = raw2 * isd
  agg2 = _agg64(g2, src4, dst4, z32)

  return (isd * agg2 + b2)[:N]
